# baseline (speedup 1.0000x reference)
import functools
import math

import jax
import jax.numpy as jnp
import numpy as np
from jax.experimental import pallas as pl
from jax.experimental.pallas import tpu as pltpu

N_NODES = 10000
RATIO = 0.5

NP = 10240
K1, K2, K3 = 5000, 2500, 1250
K1P, K2P, K3P = 5120, 2560, 1280


def _pick(M, prefs):
    for p in prefs:
        if M % p == 0:
            return p
    raise ValueError(f"no block size for {M}")

def _mm_pool_body(r_ref, s_ref, o_ref, rs_ref, acc_ref, *, ksteps, bm, bn):
    mi = pl.program_id(0)
    ni = pl.program_id(1)
    ki = pl.program_id(2)

    @pl.when(ki == 0)
    def _():
        acc_ref[...] = jnp.zeros_like(acc_ref)

    acc_ref[...] += jnp.dot(r_ref[...], s_ref[...],
                            preferred_element_type=jnp.float32)

    @pl.when(ki == ksteps - 1)
    def _():
        acc = acc_ref[...]
        rows = mi * bm + jax.lax.broadcasted_iota(jnp.int32, (bm, bn), 0)
        cols = ni * bn + jax.lax.broadcasted_iota(jnp.int32, (bm, bn), 1)
        acc = jnp.where(rows == cols, 0.0, acc)
        o_ref[...] = acc

        @pl.when(ni == 0)
        def _():
            rs_ref[...] = jnp.zeros_like(rs_ref)

        rs_ref[...] += jnp.sum(acc, axis=1, keepdims=True)


def _mm_pool(R, S, bm=None, bn=None, bk=None):
    M, K = R.shape
    K2_, N = S.shape
    bm = bm or _pick(M, (512, 256, 128))
    bn = bn or _pick(N, (512, 256, 128))
    bk = bk or _pick(K, (512, 256, 128))
    assert K == K2_ and M % bm == 0 and N % bn == 0 and K % bk == 0
    grid = (M // bm, N // bn, K // bk)
    return pl.pallas_call(
        functools.partial(_mm_pool_body, ksteps=K // bk, bm=bm, bn=bn),
        grid=grid,
        in_specs=[
            pl.BlockSpec((bm, bk), lambda m, n, k: (m, k)),
            pl.BlockSpec((bk, bn), lambda m, n, k: (k, n)),
        ],
        out_specs=[
            pl.BlockSpec((bm, bn), lambda m, n, k: (m, n)),
            pl.BlockSpec((bm, 1), lambda m, n, k: (m, 0)),
        ],
        out_shape=[
            jax.ShapeDtypeStruct((M, N), jnp.float32),
            jax.ShapeDtypeStruct((M, 1), jnp.float32),
        ],
        scratch_shapes=[pltpu.VMEM((bm, bn), jnp.float32)],
        compiler_params=pltpu.CompilerParams(
            dimension_semantics=("parallel", "parallel", "arbitrary")),
    )(R, S)


def _mm_agg_body(a_ref, u_ref, um_ref, dinv_ref, b_ref, o_ref, acc_ref,
                 *, ksteps, relu):
    ki = pl.program_id(1)

    @pl.when(ki == 0)
    def _():
        acc_ref[...] = jnp.zeros_like(acc_ref)

    acc_ref[...] += jnp.dot(a_ref[...], u_ref[...],
                            preferred_element_type=jnp.float32)

    @pl.when(ki == ksteps - 1)
    def _():
        out = dinv_ref[...] * (acc_ref[...] + 2.0 * um_ref[...]) + b_ref[...]
        if relu:
            out = jnp.maximum(out, 0.0)
        o_ref[...] = out


def _mm_agg(A, u, dinv, b, relu, bm=None, bk=None):
    M, C = u.shape
    bm = bm or _pick(M, (512, 256, 128))
    bk = bk or _pick(M, (512, 256, 128))
    assert A.shape == (M, M) and M % bm == 0 and M % bk == 0
    grid = (M // bm, M // bk)
    return pl.pallas_call(
        functools.partial(_mm_agg_body, ksteps=M // bk, relu=relu),
        grid=grid,
        in_specs=[
            pl.BlockSpec((bm, bk), lambda m, k: (m, k)),
            pl.BlockSpec((bk, C), lambda m, k: (k, 0)),
            pl.BlockSpec((bm, C), lambda m, k: (m, 0)),
            pl.BlockSpec((bm, 1), lambda m, k: (m, 0)),
            pl.BlockSpec((1, C), lambda m, k: (0, 0)),
        ],
        out_specs=pl.BlockSpec((bm, C), lambda m, k: (m, 0)),
        out_shape=jax.ShapeDtypeStruct((M, C), jnp.float32),
        scratch_shapes=[pltpu.VMEM((bm, C), jnp.float32)],
        compiler_params=pltpu.CompilerParams(
            dimension_semantics=("parallel", "arbitrary")),
    )(A, u, u, dinv, b)


def _mm_xw_body(x_ref, w_ref, dinv_ref, o_ref):
    o_ref[...] = dinv_ref[...] * jnp.dot(x_ref[...], w_ref[...],
                                         preferred_element_type=jnp.float32)


def _mm_xw(x, W, dinv, bm=None):
    M, K = x.shape
    C = W.shape[1]
    bm = bm or _pick(M, (1024, 512, 256, 128))
    assert M % bm == 0
    return pl.pallas_call(
        _mm_xw_body,
        grid=(M // bm,),
        in_specs=[
            pl.BlockSpec((bm, K), lambda m: (m, 0)),
            pl.BlockSpec((K, C), lambda m: (0, 0)),
            pl.BlockSpec((bm, 1), lambda m: (m, 0)),
        ],
        out_specs=pl.BlockSpec((bm, C), lambda m: (m, 0)),
        out_shape=jax.ShapeDtypeStruct((M, C), jnp.float32),
        compiler_params=pltpu.CompilerParams(
            dimension_semantics=("parallel",)),
    )(x, W, dinv)


def _pad_rows(a, rows):
    return jnp.pad(a, ((0, rows - a.shape[0]), (0, 0)))


def _score_topk(h, p, n_valid, k):
    s = jnp.tanh((h[:n_valid] @ p) / jnp.linalg.norm(p))
    _, perm = jax.lax.top_k(s, k)
    return s, perm


def _pool_level(Acur_p, h_p, s, perm, n_valid, np_, kp):
    k = perm.shape[0]
    B = Acur_p.at[jnp.arange(n_valid), jnp.arange(n_valid)].set(1.0)
    Rb = _pad_rows(B[perm, :], kp).astype(jnp.bfloat16)
    Sb = jnp.pad(B[:, perm], ((0, 0), (0, kp - k))).astype(jnp.bfloat16)
    A_next, rs = _mm_pool(Rb, Sb)
    dinv = (rs + 2.0) ** -0.5
    hp = _pad_rows(h_p[perm] * s[perm][:, None], kp)
    return A_next, dinv, hp


def kernel(x, edge_index, W0, b0, W1, b1, W2, b2, W3, b3,
           p1, p2, p3, U0, ub0, U1, ub1, U2, ub2):
    n = N_NODES
    ei = edge_index.astype(jnp.int32)
    dst, src = ei[1], ei[0]

    deg0 = jnp.zeros((n,), jnp.float32).at[dst].add(1.0) + 2.0
    dinv0 = deg0 ** -0.5
    dinv0_p = _pad_rows(dinv0[:, None], NP)
    x_p = _pad_rows(x, NP)

    def gcn0(h_p, W, b, relu):
        u_p = _mm_xw(h_p, W, dinv0_p)
        agg = jnp.zeros_like(u_p).at[dst].add(u_p[src])
        out = dinv0_p * (agg + 2.0 * u_p) + b[None, :]
        return jnp.maximum(out, 0.0) if relu else out

    h0_p = gcn0(x_p, W0, b0, True)

    s1, perm1 = _score_topk(h0_p, p1, n, K1)
    inv1 = jnp.full((n,), -1, jnp.int32).at[perm1].set(
        jnp.arange(K1, dtype=jnp.int32))
    nd = dst != src
    r_rows = jnp.where(nd, inv1[dst], -1)
    R = jnp.zeros((K1P, NP), jnp.float32).at[r_rows, src].add(
        jnp.where(r_rows >= 0, 1.0, 0.0), mode="drop")
    R = R.at[jnp.arange(K1), perm1].add(1.0)
    s_cols = jnp.where(nd, inv1[src], -1)
    S = jnp.zeros((NP, K1P), jnp.float32).at[dst, s_cols].add(
        jnp.where(s_cols >= 0, 1.0, 0.0), mode="drop")
    S = S.at[perm1, jnp.arange(K1)].add(1.0)
    A1, rs1 = _mm_pool(R.astype(jnp.bfloat16), S.astype(jnp.bfloat16))
    dinv1 = (rs1 + 2.0) ** -0.5
    h1p = _pad_rows(h0_p[perm1] * s1[perm1][:, None], K1P)

    def gcn(h_p, A_p, dinv_p, W, b, relu):
        u_p = _mm_xw(h_p, W, dinv_p)
        return _mm_agg(A_p, u_p, dinv_p, b[None, :], relu)

    h1 = gcn(h1p, A1, dinv1, W1, b1, True)

    s2, perm2 = _score_topk(h1, p2, K1, K2)
    A2, dinv2, h2p = _pool_level(A1, h1, s2, perm2, K1, K1P, K2P)
    h2 = gcn(h2p, A2, dinv2, W2, b2, True)

    s3, perm3 = _score_topk(h2, p3, K2, K3)
    A3, dinv3, h3p = _pool_level(A2, h2, s3, perm3, K2, K2P, K3P)
    h = gcn(h3p, A3, dinv3, W3, b3, True)

    h = h2.at[perm3].add(h[:K3])
    h = gcn(h, A2, dinv2, U0, ub0, True)
    h = h1.at[perm2].add(h[:K2])
    h = gcn(h, A1, dinv1, U1, ub1, True)
    h = h0_p.at[perm1].add(h[:K1])
    h = gcn0(h, U2, ub2, False)
    return h[:n]
